# baseline (speedup 1.0000x reference)
import functools

import jax
import jax.numpy as jnp
from jax import lax
from jax.experimental import pallas as pl
from jax.experimental.pallas import tpu as pltpu
from jax.experimental.pallas import tpu_sc as plsc

N = 10000
E = 320000
H = 128
HIN = 144

GROUPS = (144000, 144000, 32000)
STARTS = (0, 144000, 288000)


BE = 3200


def _edge_mlp_body(he_ref, w1_ref, b1_ref, w2_ref, b2_ref, w3_ref, b3_ref, out_ref):
    xt = he_ref[...]
    h1 = jnp.maximum(
        lax.dot_general(xt, w1_ref[...], (((0,), (0,)), ((), ())),
                        preferred_element_type=jnp.float32) + b1_ref[...], 0.0)
    h2 = jnp.maximum(jnp.dot(h1, w2_ref[...], preferred_element_type=jnp.float32) + b2_ref[...], 0.0)
    y = jnp.dot(h2, w3_ref[...], preferred_element_type=jnp.float32) + b3_ref[...]
    out_ref[...] = y * (1.0 / 30.0)


def _edge_mlp(h_ET, W1w, W1b, W2w, W2b, W3w, W3b, start, size):
    blk0 = start // BE
    return pl.pallas_call(
        _edge_mlp_body,
        grid=(size // BE,),
        in_specs=[
            pl.BlockSpec((HIN, BE), lambda i: (0, blk0 + i)),
            pl.BlockSpec((HIN, H), lambda i: (0, 0)),
            pl.BlockSpec((1, H), lambda i: (0, 0)),
            pl.BlockSpec((H, H), lambda i: (0, 0)),
            pl.BlockSpec((1, H), lambda i: (0, 0)),
            pl.BlockSpec((H, H), lambda i: (0, 0)),
            pl.BlockSpec((1, H), lambda i: (0, 0)),
        ],
        out_specs=pl.BlockSpec((BE, H), lambda i: (i, 0)),
        out_shape=jax.ShapeDtypeStruct((size, H), jnp.float32),
    )(h_ET, W1w, W1b, W2w, W2b, W3w, W3b)


NC = 2
NS = 16
NW = NC * NS
CH = 128
CZ = 80
NCHUNKS = N // CZ


def _scatter_partials(m, edge_idx, zero, start, size):
    mesh = plsc.VectorSubcoreMesh(core_axis_name="c", subcore_axis_name="s")
    nch = size // CH
    cbase = nch // NW
    extra = nch % NW

    @functools.partial(
        pl.kernel,
        mesh=mesh,
        out_type=jax.ShapeDtypeStruct((NC, N, H), jnp.float32),
        scratch_types=[
            pltpu.VMEM((2, CH), jnp.int32),
            pltpu.VMEM((CH, H), jnp.float32),
            pltpu.VMEM((2, CH), jnp.int32),
            pltpu.VMEM((CH, H), jnp.float32),
            pltpu.VMEM((CZ, H), jnp.float32),
            pltpu.VMEM_SHARED((N, H), jnp.float32),
            pltpu.SemaphoreType.DMA,
            pltpu.SemaphoreType.DMA,
            pltpu.SemaphoreType.DMA,
            pltpu.SemaphoreType.DMA,
        ],
    )
    def sc_kernel(m_hbm, idx_hbm, zero_hbm, out_hbm,
                  ibuf_a, mbuf_a, ibuf_b, mbuf_b, zbuf, acc,
                  sem_a, sem_b, sem_sa, sem_sb):
        c = lax.axis_index("c")
        s = lax.axis_index("s")
        nk = lax.select(s < NCHUNKS % NS, NCHUNKS // NS + 1, NCHUNKS // NS)
        pltpu.sync_copy(zero_hbm, zbuf)

        def zero_body(k, carry):
            pltpu.sync_copy(zbuf, acc.at[pl.ds((s + k * NS) * CZ, CZ)])
            return carry

        lax.fori_loop(0, nk, zero_body, 0)
        plsc.subcore_barrier()
        w = c * NS + s
        mycnt = lax.select(w < extra, cbase + 1, cbase)
        my0 = w * cbase + jnp.minimum(w, extra)

        def load(ch, ib, mb, sem):
            pltpu.async_copy(idx_hbm.at[:, pl.ds(start + ch * CH, CH)], ib, sem)
            pltpu.async_copy(m_hbm.at[pl.ds(ch * CH, CH)], mb, sem)

        def wait(ib, mb, sem):
            pltpu.make_async_copy(idx_hbm.at[:, pl.ds(0, CH)], ib, sem).wait()
            pltpu.make_async_copy(m_hbm.at[pl.ds(0, CH)], mb, sem).wait()

        load(my0, ibuf_a, mbuf_a, sem_a)

        def body(j, carry):
            k = 2 * j
            wait(ibuf_a, mbuf_a, sem_a)

            @pl.when(j > 0)
            def _():
                pltpu.make_async_copy(mbuf_b, acc.at[ibuf_b.at[0]], sem_sb).wait()

            load(my0 + k + 1, ibuf_b, mbuf_b, sem_b)
            pltpu.async_copy(mbuf_a, acc.at[ibuf_a.at[0]], sem_sa, add=True)
            wait(ibuf_b, mbuf_b, sem_b)
            pltpu.make_async_copy(mbuf_a, acc.at[ibuf_a.at[0]], sem_sa).wait()

            @pl.when(k + 2 < mycnt)
            def _():
                load(my0 + k + 2, ibuf_a, mbuf_a, sem_a)

            pltpu.async_copy(mbuf_b, acc.at[ibuf_b.at[0]], sem_sb, add=True)
            return carry

        lax.fori_loop(0, mycnt // 2, body, 0)
        pltpu.make_async_copy(mbuf_b, acc.at[ibuf_b.at[0]], sem_sb).wait()

        @pl.when(mycnt % 2 == 1)
        def _():
            wait(ibuf_a, mbuf_a, sem_a)
            pltpu.sync_copy(mbuf_a, acc.at[ibuf_a.at[0]], add=True)

        plsc.subcore_barrier()
        def out_body(k, carry):
            sl = pl.ds((s + k * NS) * CZ, CZ)
            pltpu.sync_copy(acc.at[sl], out_hbm.at[c, sl])
            return carry

        lax.fori_loop(0, nk, out_body, 0)

    return sc_kernel(m, edge_idx, zero)


BN = 2000
NPART = len(GROUPS) * NC


def _node_body(*refs):
    hv_ref = refs[0]
    parts = refs[1:1 + NPART]
    d1w_ref, d1b_ref, d2w_ref, d2b_ref, g1_ref, b1_ref, g2_ref, b2_ref = refs[1 + NPART:-1]
    out_ref = refs[-1]
    x = hv_ref[...]
    for p in parts:
        x = x + p[0]
    mu = jnp.mean(x, axis=-1, keepdims=True)
    xc = x - mu
    var = jnp.mean(xc * xc, axis=-1, keepdims=True)
    hv1 = xc * lax.rsqrt(var + 1e-5) * g1_ref[...] + b1_ref[...]
    t = jnp.maximum(jnp.dot(hv1, d1w_ref[...], preferred_element_type=jnp.float32) + d1b_ref[...], 0.0)
    x2 = hv1 + jnp.dot(t, d2w_ref[...], preferred_element_type=jnp.float32) + d2b_ref[...]
    mu2 = jnp.mean(x2, axis=-1, keepdims=True)
    xc2 = x2 - mu2
    var2 = jnp.mean(xc2 * xc2, axis=-1, keepdims=True)
    out_ref[...] = xc2 * lax.rsqrt(var2 + 1e-5) * g2_ref[...] + b2_ref[...]


def _node_update(h_V, partials, d1w, d1b, d2w, d2b, g1, b1, g2, b2):
    full = lambda shape: pl.BlockSpec(shape, lambda i: tuple(0 for _ in shape))
    node_blk = pl.BlockSpec((BN, H), lambda i: (i, 0))
    flat_parts = []
    part_specs = []
    for p in partials:
        for cidx in range(NC):
            flat_parts.append(p)
            part_specs.append(pl.BlockSpec((1, BN, H), lambda i, c=cidx: (c, i, 0)))
    return pl.pallas_call(
        _node_body,
        grid=(N // BN,),
        in_specs=[node_blk] + part_specs + [
            full((H, 4 * H)),
            full((1, 4 * H)),
            full((4 * H, H)),
            full((1, H)),
            full((1, H)),
            full((1, H)),
            full((1, H)),
            full((1, H)),
        ],
        out_specs=node_blk,
        out_shape=jax.ShapeDtypeStruct((N, H), jnp.float32),
    )(h_V, *flat_parts, d1w, d1b, d2w, d2b, g1, b1, g2, b2)


def kernel(h_V, h_E, edge_idx, W1w, W1b, W2w, W2b, W3w, W3b, g1, b1, d1w, d1b, d2w, d2b, g2, b2):
    row = lambda v: v.reshape(1, -1)
    zero = jnp.zeros((CZ, H), jnp.float32)
    h_ET = h_E.T
    partials = []
    for start, size in zip(STARTS, GROUPS):
        m_k = _edge_mlp(h_ET, W1w, row(W1b), W2w, row(W2b), W3w, row(W3b), start, size)
        partials.append(_scatter_partials(m_k, edge_idx, zero, start, size))
    return _node_update(h_V, partials, d1w, row(d1b), d2w, row(d2b),
                        row(g1), row(b1), row(g2), row(b2))


# speedup vs baseline: 1.0589x; 1.0589x over previous
import functools

import jax
import jax.numpy as jnp
from jax import lax
from jax.experimental import pallas as pl
from jax.experimental.pallas import tpu as pltpu
from jax.experimental.pallas import tpu_sc as plsc

N = 10000
E = 320000
H = 128
HIN = 144

GROUPS = (144000, 144000, 32000)
STARTS = (0, 144000, 288000)
BES = (9600, 9600, 6400)


def _edge_mlp_body(he_ref, w1_ref, b1_ref, w2_ref, b2_ref, w3_ref, b3_ref, out_ref):
    xt = he_ref[...]
    h1 = jnp.maximum(
        lax.dot_general(xt, w1_ref[...], (((0,), (0,)), ((), ())),
                        preferred_element_type=jnp.float32) + b1_ref[...], 0.0)
    h2 = jnp.maximum(jnp.dot(h1, w2_ref[...], preferred_element_type=jnp.float32) + b2_ref[...], 0.0)
    y = jnp.dot(h2, w3_ref[...], preferred_element_type=jnp.float32) + b3_ref[...]
    out_ref[...] = y * (1.0 / 30.0)


def _edge_mlp(h_ET, W1w, W1b, W2w, W2b, W3w, W3b, start, size, be):
    assert start % be == 0 and size % be == 0 and be % 128 == 0
    blk0 = start // be
    return pl.pallas_call(
        _edge_mlp_body,
        grid=(size // be,),
        in_specs=[
            pl.BlockSpec((HIN, be), lambda i: (0, blk0 + i)),
            pl.BlockSpec((HIN, H), lambda i: (0, 0)),
            pl.BlockSpec((1, H), lambda i: (0, 0)),
            pl.BlockSpec((H, H), lambda i: (0, 0)),
            pl.BlockSpec((1, H), lambda i: (0, 0)),
            pl.BlockSpec((H, H), lambda i: (0, 0)),
            pl.BlockSpec((1, H), lambda i: (0, 0)),
        ],
        out_specs=pl.BlockSpec((be, H), lambda i: (i, 0)),
        out_shape=jax.ShapeDtypeStruct((size, H), jnp.float32),
    )(h_ET, W1w, W1b, W2w, W2b, W3w, W3b)


NC = 2
NS = 16
NW = NC * NS
CH = 128
CZ = 80
NCHUNKS = N // CZ


def _scatter_partials(m, edge_idx, zero, start, size):
    mesh = plsc.VectorSubcoreMesh(core_axis_name="c", subcore_axis_name="s")
    nch = size // CH
    cbase = nch // NW
    extra = nch % NW

    @functools.partial(
        pl.kernel,
        mesh=mesh,
        out_type=jax.ShapeDtypeStruct((NC, N, H), jnp.float32),
        scratch_types=[
            pltpu.VMEM((2, CH), jnp.int32),
            pltpu.VMEM((CH, H), jnp.float32),
            pltpu.VMEM((2, CH), jnp.int32),
            pltpu.VMEM((CH, H), jnp.float32),
            pltpu.VMEM((CZ, H), jnp.float32),
            pltpu.VMEM_SHARED((N, H), jnp.float32),
            pltpu.SemaphoreType.DMA,
            pltpu.SemaphoreType.DMA,
            pltpu.SemaphoreType.DMA,
            pltpu.SemaphoreType.DMA,
        ],
    )
    def sc_kernel(m_hbm, idx_hbm, zero_hbm, out_hbm,
                  ibuf_a, mbuf_a, ibuf_b, mbuf_b, zbuf, acc,
                  sem_a, sem_b, sem_sa, sem_sb):
        c = lax.axis_index("c")
        s = lax.axis_index("s")
        nk = lax.select(s < NCHUNKS % NS, NCHUNKS // NS + 1, NCHUNKS // NS)
        pltpu.sync_copy(zero_hbm, zbuf)

        def zero_body(k, carry):
            pltpu.sync_copy(zbuf, acc.at[pl.ds((s + k * NS) * CZ, CZ)])
            return carry

        lax.fori_loop(0, nk, zero_body, 0)
        plsc.subcore_barrier()
        w = c * NS + s
        mycnt = lax.select(w < extra, cbase + 1, cbase)
        my0 = w * cbase + jnp.minimum(w, extra)

        def load(ch, ib, mb, sem):
            pltpu.async_copy(idx_hbm.at[:, pl.ds(start + ch * CH, CH)], ib, sem)
            pltpu.async_copy(m_hbm.at[pl.ds(ch * CH, CH)], mb, sem)

        def wait(ib, mb, sem):
            pltpu.make_async_copy(idx_hbm.at[:, pl.ds(0, CH)], ib, sem).wait()
            pltpu.make_async_copy(m_hbm.at[pl.ds(0, CH)], mb, sem).wait()

        load(my0, ibuf_a, mbuf_a, sem_a)

        def body(j, carry):
            k = 2 * j
            wait(ibuf_a, mbuf_a, sem_a)

            @pl.when(j > 0)
            def _():
                pltpu.make_async_copy(mbuf_b, acc.at[ibuf_b.at[0]], sem_sb).wait()

            load(my0 + k + 1, ibuf_b, mbuf_b, sem_b)
            pltpu.async_copy(mbuf_a, acc.at[ibuf_a.at[0]], sem_sa, add=True)
            wait(ibuf_b, mbuf_b, sem_b)
            pltpu.make_async_copy(mbuf_a, acc.at[ibuf_a.at[0]], sem_sa).wait()

            @pl.when(k + 2 < mycnt)
            def _():
                load(my0 + k + 2, ibuf_a, mbuf_a, sem_a)

            pltpu.async_copy(mbuf_b, acc.at[ibuf_b.at[0]], sem_sb, add=True)
            return carry

        lax.fori_loop(0, mycnt // 2, body, 0)
        pltpu.make_async_copy(mbuf_b, acc.at[ibuf_b.at[0]], sem_sb).wait()

        @pl.when(mycnt % 2 == 1)
        def _():
            wait(ibuf_a, mbuf_a, sem_a)
            pltpu.sync_copy(mbuf_a, acc.at[ibuf_a.at[0]], add=True)

        plsc.subcore_barrier()
        def out_body(k, carry):
            sl = pl.ds((s + k * NS) * CZ, CZ)
            pltpu.sync_copy(acc.at[sl], out_hbm.at[c, sl])
            return carry

        lax.fori_loop(0, nk, out_body, 0)

    return sc_kernel(m, edge_idx, zero)


BN = 2000
NPART = len(GROUPS) * NC


def _node_body(*refs):
    hv_ref = refs[0]
    parts = refs[1:1 + NPART]
    d1w_ref, d1b_ref, d2w_ref, d2b_ref, g1_ref, b1_ref, g2_ref, b2_ref = refs[1 + NPART:-1]
    out_ref = refs[-1]
    x = hv_ref[...]
    for p in parts:
        x = x + p[0]
    mu = jnp.mean(x, axis=-1, keepdims=True)
    xc = x - mu
    var = jnp.mean(xc * xc, axis=-1, keepdims=True)
    hv1 = xc * lax.rsqrt(var + 1e-5) * g1_ref[...] + b1_ref[...]
    t = jnp.maximum(jnp.dot(hv1, d1w_ref[...], preferred_element_type=jnp.float32) + d1b_ref[...], 0.0)
    x2 = hv1 + jnp.dot(t, d2w_ref[...], preferred_element_type=jnp.float32) + d2b_ref[...]
    mu2 = jnp.mean(x2, axis=-1, keepdims=True)
    xc2 = x2 - mu2
    var2 = jnp.mean(xc2 * xc2, axis=-1, keepdims=True)
    out_ref[...] = xc2 * lax.rsqrt(var2 + 1e-5) * g2_ref[...] + b2_ref[...]


def _node_update(h_V, partials, d1w, d1b, d2w, d2b, g1, b1, g2, b2):
    full = lambda shape: pl.BlockSpec(shape, lambda i: tuple(0 for _ in shape))
    node_blk = pl.BlockSpec((BN, H), lambda i: (i, 0))
    flat_parts = []
    part_specs = []
    for p in partials:
        for cidx in range(NC):
            flat_parts.append(p)
            part_specs.append(pl.BlockSpec((1, BN, H), lambda i, c=cidx: (c, i, 0)))
    return pl.pallas_call(
        _node_body,
        grid=(N // BN,),
        in_specs=[node_blk] + part_specs + [
            full((H, 4 * H)),
            full((1, 4 * H)),
            full((4 * H, H)),
            full((1, H)),
            full((1, H)),
            full((1, H)),
            full((1, H)),
            full((1, H)),
        ],
        out_specs=node_blk,
        out_shape=jax.ShapeDtypeStruct((N, H), jnp.float32),
    )(h_V, *flat_parts, d1w, d1b, d2w, d2b, g1, b1, g2, b2)


def kernel(h_V, h_E, edge_idx, W1w, W1b, W2w, W2b, W3w, W3b, g1, b1, d1w, d1b, d2w, d2b, g2, b2):
    row = lambda v: v.reshape(1, -1)
    zero = jnp.zeros((CZ, H), jnp.float32)
    h_ET = h_E.T
    partials = []
    for start, size, be in zip(STARTS, GROUPS, BES):
        m_k = _edge_mlp(h_ET, W1w, row(W1b), W2w, row(W2b), W3w, row(W3b), start, size, be)
        partials.append(_scatter_partials(m_k, edge_idx, zero, start, size))
    return _node_update(h_V, partials, d1w, row(d1b), d2w, row(d2b),
                        row(g1), row(b1), row(g2), row(b2))


# speedup vs baseline: 1.0772x; 1.0173x over previous
import functools

import jax
import jax.numpy as jnp
from jax import lax
from jax.experimental import pallas as pl
from jax.experimental.pallas import tpu as pltpu
from jax.experimental.pallas import tpu_sc as plsc

N = 10000
E = 320000
H = 128
HIN = 144

GROUPS = (128000, 128000, 64000)
STARTS = (0, 128000, 256000)


BE = 12800


def _edge_mlp_body(he_ref, w1_ref, b1_ref, w2_ref, b2_ref, w3_ref, b3_ref, out_ref):
    xt = he_ref[...]
    h1 = jnp.maximum(
        lax.dot_general(xt, w1_ref[...], (((0,), (0,)), ((), ())),
                        preferred_element_type=jnp.float32) + b1_ref[...], 0.0)
    h2 = jnp.maximum(jnp.dot(h1, w2_ref[...], preferred_element_type=jnp.float32) + b2_ref[...], 0.0)
    y = jnp.dot(h2, w3_ref[...], preferred_element_type=jnp.float32) + b3_ref[...]
    out_ref[...] = y * (1.0 / 30.0)


def _edge_mlp(h_ET, W1w, W1b, W2w, W2b, W3w, W3b, start, size):
    blk0 = start // BE
    return pl.pallas_call(
        _edge_mlp_body,
        grid=(size // BE,),
        in_specs=[
            pl.BlockSpec((HIN, BE), lambda i: (0, blk0 + i)),
            pl.BlockSpec((HIN, H), lambda i: (0, 0)),
            pl.BlockSpec((1, H), lambda i: (0, 0)),
            pl.BlockSpec((H, H), lambda i: (0, 0)),
            pl.BlockSpec((1, H), lambda i: (0, 0)),
            pl.BlockSpec((H, H), lambda i: (0, 0)),
            pl.BlockSpec((1, H), lambda i: (0, 0)),
        ],
        out_specs=pl.BlockSpec((BE, H), lambda i: (i, 0)),
        out_shape=jax.ShapeDtypeStruct((size, H), jnp.float32),
    )(h_ET, W1w, W1b, W2w, W2b, W3w, W3b)


NC = 2
NS = 16
NW = NC * NS
CH = 128
CZ = 80
NCHUNKS = N // CZ


def _scatter_partials(m, edge_idx, zero, start, size):
    mesh = plsc.VectorSubcoreMesh(core_axis_name="c", subcore_axis_name="s")
    nch = size // CH
    cbase = nch // NW
    extra = nch % NW

    @functools.partial(
        pl.kernel,
        mesh=mesh,
        out_type=jax.ShapeDtypeStruct((NC, N, H), jnp.float32),
        scratch_types=[
            pltpu.VMEM((2, CH), jnp.int32),
            pltpu.VMEM((CH, H), jnp.float32),
            pltpu.VMEM((2, CH), jnp.int32),
            pltpu.VMEM((CH, H), jnp.float32),
            pltpu.VMEM((CZ, H), jnp.float32),
            pltpu.VMEM_SHARED((N, H), jnp.float32),
            pltpu.SemaphoreType.DMA,
            pltpu.SemaphoreType.DMA,
            pltpu.SemaphoreType.DMA,
            pltpu.SemaphoreType.DMA,
        ],
    )
    def sc_kernel(m_hbm, idx_hbm, zero_hbm, out_hbm,
                  ibuf_a, mbuf_a, ibuf_b, mbuf_b, zbuf, acc,
                  sem_a, sem_b, sem_sa, sem_sb):
        c = lax.axis_index("c")
        s = lax.axis_index("s")
        nk = lax.select(s < NCHUNKS % NS, NCHUNKS // NS + 1, NCHUNKS // NS)
        pltpu.sync_copy(zero_hbm, zbuf)

        def zero_body(k, carry):
            pltpu.sync_copy(zbuf, acc.at[pl.ds((s + k * NS) * CZ, CZ)])
            return carry

        lax.fori_loop(0, nk, zero_body, 0)
        plsc.subcore_barrier()
        w = c * NS + s
        mycnt = lax.select(w < extra, cbase + 1, cbase)
        my0 = w * cbase + jnp.minimum(w, extra)

        def load(ch, ib, mb, sem):
            pltpu.async_copy(idx_hbm.at[:, pl.ds(start + ch * CH, CH)], ib, sem)
            pltpu.async_copy(m_hbm.at[pl.ds(ch * CH, CH)], mb, sem)

        def wait(ib, mb, sem):
            pltpu.make_async_copy(idx_hbm.at[:, pl.ds(0, CH)], ib, sem).wait()
            pltpu.make_async_copy(m_hbm.at[pl.ds(0, CH)], mb, sem).wait()

        load(my0, ibuf_a, mbuf_a, sem_a)

        def body(j, carry):
            k = 2 * j
            wait(ibuf_a, mbuf_a, sem_a)

            @pl.when(j > 0)
            def _():
                pltpu.make_async_copy(mbuf_b, acc.at[ibuf_b.at[0]], sem_sb).wait()

            load(my0 + k + 1, ibuf_b, mbuf_b, sem_b)
            pltpu.async_copy(mbuf_a, acc.at[ibuf_a.at[0]], sem_sa, add=True)
            wait(ibuf_b, mbuf_b, sem_b)
            pltpu.make_async_copy(mbuf_a, acc.at[ibuf_a.at[0]], sem_sa).wait()

            @pl.when(k + 2 < mycnt)
            def _():
                load(my0 + k + 2, ibuf_a, mbuf_a, sem_a)

            pltpu.async_copy(mbuf_b, acc.at[ibuf_b.at[0]], sem_sb, add=True)
            return carry

        lax.fori_loop(0, mycnt // 2, body, 0)
        pltpu.make_async_copy(mbuf_b, acc.at[ibuf_b.at[0]], sem_sb).wait()

        @pl.when(mycnt % 2 == 1)
        def _():
            wait(ibuf_a, mbuf_a, sem_a)
            pltpu.sync_copy(mbuf_a, acc.at[ibuf_a.at[0]], add=True)

        plsc.subcore_barrier()
        def out_body(k, carry):
            sl = pl.ds((s + k * NS) * CZ, CZ)
            pltpu.sync_copy(acc.at[sl], out_hbm.at[c, sl])
            return carry

        lax.fori_loop(0, nk, out_body, 0)

    return sc_kernel(m, edge_idx, zero)


BN = 2000
NPART = len(GROUPS) * NC


def _node_body(*refs):
    hv_ref = refs[0]
    parts = refs[1:1 + NPART]
    d1w_ref, d1b_ref, d2w_ref, d2b_ref, g1_ref, b1_ref, g2_ref, b2_ref = refs[1 + NPART:-1]
    out_ref = refs[-1]
    x = hv_ref[...]
    for p in parts:
        x = x + p[0]
    mu = jnp.mean(x, axis=-1, keepdims=True)
    xc = x - mu
    var = jnp.mean(xc * xc, axis=-1, keepdims=True)
    hv1 = xc * lax.rsqrt(var + 1e-5) * g1_ref[...] + b1_ref[...]
    t = jnp.maximum(jnp.dot(hv1, d1w_ref[...], preferred_element_type=jnp.float32) + d1b_ref[...], 0.0)
    x2 = hv1 + jnp.dot(t, d2w_ref[...], preferred_element_type=jnp.float32) + d2b_ref[...]
    mu2 = jnp.mean(x2, axis=-1, keepdims=True)
    xc2 = x2 - mu2
    var2 = jnp.mean(xc2 * xc2, axis=-1, keepdims=True)
    out_ref[...] = xc2 * lax.rsqrt(var2 + 1e-5) * g2_ref[...] + b2_ref[...]


def _node_update(h_V, partials, d1w, d1b, d2w, d2b, g1, b1, g2, b2):
    full = lambda shape: pl.BlockSpec(shape, lambda i: tuple(0 for _ in shape))
    node_blk = pl.BlockSpec((BN, H), lambda i: (i, 0))
    flat_parts = []
    part_specs = []
    for p in partials:
        for cidx in range(NC):
            flat_parts.append(p)
            part_specs.append(pl.BlockSpec((1, BN, H), lambda i, c=cidx: (c, i, 0)))
    return pl.pallas_call(
        _node_body,
        grid=(N // BN,),
        in_specs=[node_blk] + part_specs + [
            full((H, 4 * H)),
            full((1, 4 * H)),
            full((4 * H, H)),
            full((1, H)),
            full((1, H)),
            full((1, H)),
            full((1, H)),
            full((1, H)),
        ],
        out_specs=node_blk,
        out_shape=jax.ShapeDtypeStruct((N, H), jnp.float32),
    )(h_V, *flat_parts, d1w, d1b, d2w, d2b, g1, b1, g2, b2)


def kernel(h_V, h_E, edge_idx, W1w, W1b, W2w, W2b, W3w, W3b, g1, b1, d1w, d1b, d2w, d2b, g2, b2):
    row = lambda v: v.reshape(1, -1)
    zero = jnp.zeros((CZ, H), jnp.float32)
    h_ET = h_E.T
    partials = []
    for start, size in zip(STARTS, GROUPS):
        m_k = _edge_mlp(h_ET, W1w, row(W1b), W2w, row(W2b), W3w, row(W3b), start, size)
        partials.append(_scatter_partials(m_k, edge_idx, zero, start, size))
    return _node_update(h_V, partials, d1w, row(d1b), d2w, row(d2b),
                        row(g1), row(b1), row(g2), row(b2))
